# initial kernel scaffold (unmeasured)
import jax
import jax.numpy as jnp
from jax import lax
from jax.experimental import pallas as pl
from jax.experimental.pallas import tpu as pltpu

N_DEV = 4
B = 8
H = 8
D = 128
BS = 16
NPAGES = 512
NKEYS = NPAGES * BS
PACK = 256


def kernel(Q, K, V, bt, lens):
    lens2 = lens.reshape(B, 1)

    def body(q_ref, k_ref, v_ref, bt_ref, lens_ref, out_ref,
             gather_ref, send_sems, recv_sems):
        my_pos = lax.axis_index("i")
        left = lax.rem(my_pos - 1 + N_DEV, N_DEV)
        right = lax.rem(my_pos + 1, N_DEV)

        barrier_sem = pltpu.get_barrier_semaphore()
        for nbr in (left, right):
            pl.semaphore_signal(
                barrier_sem, inc=1,
                device_id=(nbr,), device_id_type=pl.DeviceIdType.MESH,
            )
        pl.semaphore_wait(barrier_sem, 2)

        bt_v = bt_ref[...]
        lens_v = lens_ref[...]
        base = my_pos * NPAGES

        j_idx = lax.broadcasted_iota(jnp.int32, (B, NPAGES), 1)
        valid = (j_idx < lens_v).astype(jnp.float32)
        local_id = bt_v - base
        p_iota = lax.broadcasted_iota(jnp.int32, (B, NPAGES, NPAGES), 2)
        onehot = (local_id[:, :, None] == p_iota).astype(jnp.float32)
        counts = jnp.sum(valid[:, :, None] * onehot, axis=1)
        counts_key = jnp.broadcast_to(
            counts[:, :, None], (B, NPAGES, BS)
        ).reshape(B, NKEYS)

        q = q_ref[...].reshape(B, H, D).astype(jnp.bfloat16)
        k = k_ref[...].reshape(NKEYS, H, D).astype(jnp.bfloat16)
        v = v_ref[...].reshape(NKEYS, H, D).astype(jnp.bfloat16)

        s = lax.dot_general(
            q, k, (((2,), (2,)), ((1,), (1,))),
            preferred_element_type=jnp.float32,
        ) * (D ** -0.5)

        ck = counts_key[None, :, :]
        s_masked = jnp.where(ck > 0, s, -1e30)
        m = jnp.max(s_masked, axis=-1)
        w = ck * jnp.exp(s_masked - m[:, :, None])
        l = jnp.sum(w, axis=-1)
        o = lax.dot_general(
            w.astype(jnp.bfloat16), v, (((2,), (0,)), ((0,), (1,))),
            preferred_element_type=jnp.float32,
        )

        packed = jnp.concatenate(
            [o, m[:, :, None], l[:, :, None],
             jnp.zeros((H, B, PACK - D - 2), jnp.float32)],
            axis=-1,
        )
        gather_ref[my_pos] = packed

        for h in range(N_DEV - 1):
            slot = lax.rem(my_pos - h + N_DEV, N_DEV)
            rdma = pltpu.make_async_remote_copy(
                src_ref=gather_ref.at[slot],
                dst_ref=gather_ref.at[slot],
                send_sem=send_sems.at[h],
                recv_sem=recv_sems.at[h],
                device_id=(right,),
                device_id_type=pl.DeviceIdType.MESH,
            )
            rdma.start()
            rdma.wait()

        g = gather_ref[...]
        o_all = g[:, :, :, :D]
        m_all = g[:, :, :, D]
        l_all = g[:, :, :, D + 1]
        m_g = jnp.max(m_all, axis=0)
        coef = jnp.exp(m_all - m_g[None])
        l_tot = jnp.sum(coef * l_all, axis=0)
        o_tot = jnp.sum(coef[:, :, :, None] * o_all, axis=0)
        res = o_tot / l_tot[:, :, None]
        out_ref[...] = res.transpose(1, 0, 2)[:, None, :, :]

    return pl.pallas_call(
        body,
        out_shape=jax.ShapeDtypeStruct((B, 1, H, D), jnp.float32),
        in_specs=[pl.BlockSpec(memory_space=pltpu.VMEM)] * 5,
        out_specs=pl.BlockSpec(memory_space=pltpu.VMEM),
        scratch_shapes=[
            pltpu.VMEM((N_DEV, H, B, PACK), jnp.float32),
            pltpu.SemaphoreType.DMA((N_DEV - 1,)),
            pltpu.SemaphoreType.DMA((N_DEV - 1,)),
        ],
        compiler_params=pltpu.CompilerParams(collective_id=0),
    )(Q, K, V, bt, lens2)


# baseline (device time: 195888 ns/iter reference)
import jax
import jax.numpy as jnp
from jax import lax
from jax.experimental import pallas as pl
from jax.experimental.pallas import tpu as pltpu

N_DEV = 4
B = 8
H = 8
D = 128
BS = 16
NPAGES = 512
CHUNK = 32
NSTEPS = NPAGES // CHUNK
CKEYS = CHUNK * BS
PACK = 256


def kernel(Q, K, V, bt, lens):
    lens2 = lens.reshape(B, 1)

    def body(q_ref, k_ref, v_ref, bt_ref, lens_ref, out_ref,
             o_acc, m_acc, l_acc, gather_ref, send_sems, recv_sems):
        c = pl.program_id(0)
        my_pos = lax.axis_index("i")

        @pl.when(c == 0)
        def _():
            o_acc[...] = jnp.zeros((H, B, D), jnp.float32)
            m_acc[...] = jnp.full((H, B), -1e30, jnp.float32)
            l_acc[...] = jnp.zeros((H, B), jnp.float32)

        bt_v = bt_ref[...]
        lens_v = lens_ref[...]
        base = my_pos * NPAGES + c * CHUNK

        j_idx = lax.broadcasted_iota(jnp.int32, (B, NPAGES), 1)
        valid = (j_idx < lens_v).astype(jnp.float32)
        local_id = bt_v - base
        p_iota = lax.broadcasted_iota(jnp.int32, (B, NPAGES, CHUNK), 2)
        onehot = (local_id[:, :, None] == p_iota).astype(jnp.float32)
        counts = jnp.sum(valid[:, :, None] * onehot, axis=1)
        counts_key = jnp.broadcast_to(
            counts[:, :, None], (B, CHUNK, BS)
        ).reshape(B, CKEYS)

        q = q_ref[...].reshape(B, H, D).astype(jnp.bfloat16)
        k = k_ref[...].reshape(CKEYS, H, D).astype(jnp.bfloat16)
        v = v_ref[...].reshape(CKEYS, H, D).astype(jnp.bfloat16)

        s = lax.dot_general(
            q, k, (((2,), (2,)), ((1,), (1,))),
            preferred_element_type=jnp.float32,
        ) * (D ** -0.5)

        ck = counts_key[None, :, :]
        s_masked = jnp.where(ck > 0, s, -1e30)
        m_c = jnp.max(s_masked, axis=-1)
        w = ck * jnp.exp(s_masked - m_c[:, :, None])
        l_c = jnp.sum(w, axis=-1)
        o_c = lax.dot_general(
            w.astype(jnp.bfloat16), v, (((2,), (0,)), ((0,), (1,))),
            preferred_element_type=jnp.float32,
        )

        m_old = m_acc[...]
        m_new = jnp.maximum(m_old, m_c)
        sc_old = jnp.exp(m_old - m_new)
        sc_c = jnp.exp(m_c - m_new)
        m_acc[...] = m_new
        l_acc[...] = l_acc[...] * sc_old + l_c * sc_c
        o_acc[...] = o_acc[...] * sc_old[:, :, None] + o_c * sc_c[:, :, None]

        @pl.when(c == NSTEPS - 1)
        def _():
            left = lax.rem(my_pos - 1 + N_DEV, N_DEV)
            right = lax.rem(my_pos + 1, N_DEV)

            packed = jnp.concatenate(
                [o_acc[...], m_acc[...][:, :, None], l_acc[...][:, :, None],
                 jnp.zeros((H, B, PACK - D - 2), jnp.float32)],
                axis=-1,
            )
            gather_ref[my_pos] = packed

            barrier_sem = pltpu.get_barrier_semaphore()
            for nbr in (left, right):
                pl.semaphore_signal(
                    barrier_sem, inc=1,
                    device_id=(nbr,), device_id_type=pl.DeviceIdType.MESH,
                )
            pl.semaphore_wait(barrier_sem, 2)

            for h in range(N_DEV - 1):
                slot = lax.rem(my_pos - h + N_DEV, N_DEV)
                rdma = pltpu.make_async_remote_copy(
                    src_ref=gather_ref.at[slot],
                    dst_ref=gather_ref.at[slot],
                    send_sem=send_sems.at[h],
                    recv_sem=recv_sems.at[h],
                    device_id=(right,),
                    device_id_type=pl.DeviceIdType.MESH,
                )
                rdma.start()
                rdma.wait()

            g = gather_ref[...]
            o_all = g[:, :, :, :D]
            m_all = g[:, :, :, D]
            l_all = g[:, :, :, D + 1]
            m_g = jnp.max(m_all, axis=0)
            coef = jnp.exp(m_all - m_g[None])
            l_tot = jnp.sum(coef * l_all, axis=0)
            o_tot = jnp.sum(coef[:, :, :, None] * o_all, axis=0)
            res = o_tot / l_tot[:, :, None]
            out_ref[...] = res.transpose(1, 0, 2)[:, None, :, :]

    return pl.pallas_call(
        body,
        grid=(NSTEPS,),
        out_shape=jax.ShapeDtypeStruct((B, 1, H, D), jnp.float32),
        in_specs=[
            pl.BlockSpec((B, 1, H, D), lambda c: (0, 0, 0, 0)),
            pl.BlockSpec((CHUNK, BS, H, D), lambda c: (c, 0, 0, 0)),
            pl.BlockSpec((CHUNK, BS, H, D), lambda c: (c, 0, 0, 0)),
            pl.BlockSpec((B, NPAGES, ), lambda c: (0, 0)),
            pl.BlockSpec((B, 1), lambda c: (0, 0)),
        ],
        out_specs=pl.BlockSpec((B, 1, H, D), lambda c: (0, 0, 0, 0)),
        scratch_shapes=[
            pltpu.VMEM((H, B, D), jnp.float32),
            pltpu.VMEM((H, B), jnp.float32),
            pltpu.VMEM((H, B), jnp.float32),
            pltpu.VMEM((N_DEV, H, B, PACK), jnp.float32),
            pltpu.SemaphoreType.DMA((N_DEV - 1,)),
            pltpu.SemaphoreType.DMA((N_DEV - 1,)),
        ],
        compiler_params=pltpu.CompilerParams(
            collective_id=0, vmem_limit_bytes=56 * 1024 * 1024
        ),
    )(Q, K, V, bt, lens2)


# device time: 54110 ns/iter; 3.6202x vs baseline; 3.6202x over previous
import jax
import jax.numpy as jnp
from jax import lax
from jax.experimental import pallas as pl
from jax.experimental.pallas import tpu as pltpu

N_DEV = 4
B = 8
H = 8
D = 128
BS = 16
NPAGES = 512
CHUNK = 64
NSTEPS = NPAGES // CHUNK
CKEYS = CHUNK * BS
PACK = 256


def kernel(Q, K, V, bt, lens):
    lens2 = lens.reshape(B, 1)

    def body(q_ref, k_ref, v_ref, bt_ref, lens_ref, out_ref,
             o_acc, m_acc, l_acc, gather_ref, send_sems, recv_sems):
        c = pl.program_id(0)
        my_pos = lax.axis_index("i")

        @pl.when(c == 0)
        def _():
            o_acc[...] = jnp.zeros((H, B, D), jnp.float32)
            m_acc[...] = jnp.full((H, B, 1), -1e30, jnp.float32)
            l_acc[...] = jnp.zeros((H, B, 1), jnp.float32)

        bt_v = bt_ref[...]
        lens_v = lens_ref[...]
        base = my_pos * NPAGES + c * CHUNK

        j_idx = lax.broadcasted_iota(jnp.int32, (B, NPAGES), 1)
        valid = (j_idx < lens_v).astype(jnp.float32)
        local_id = bt_v - base
        p_iota = lax.broadcasted_iota(jnp.int32, (B, NPAGES, CHUNK), 2)
        onehot = (local_id[:, :, None] == p_iota).astype(jnp.float32)
        counts = jnp.sum(valid[:, :, None] * onehot, axis=1)
        counts_key = jnp.broadcast_to(
            counts[:, :, None], (B, CHUNK, BS)
        ).reshape(B, CKEYS)

        qf = q_ref[...].reshape(B, H * D)
        kf = k_ref[...].reshape(CKEYS, H * D)
        vf = v_ref[...].reshape(CKEYS, H * D)
        scale = D ** -0.5

        for h in range(H):
            q_h = qf[:, h * D:(h + 1) * D].astype(jnp.bfloat16)
            k_h = kf[:, h * D:(h + 1) * D].astype(jnp.bfloat16)
            v_h = vf[:, h * D:(h + 1) * D].astype(jnp.bfloat16)

            s_h = lax.dot_general(
                q_h, k_h, (((1,), (1,)), ((), ())),
                preferred_element_type=jnp.float32,
            ) * scale
            s_m = jnp.where(counts_key > 0, s_h, -1e30)
            m_c = jnp.max(s_m, axis=-1, keepdims=True)
            w = counts_key * jnp.exp(s_m - m_c)
            l_c = jnp.sum(w, axis=-1, keepdims=True)
            o_c = lax.dot_general(
                w.astype(jnp.bfloat16), v_h, (((1,), (0,)), ((), ())),
                preferred_element_type=jnp.float32,
            )

            m_old = m_acc[h]
            m_new = jnp.maximum(m_old, m_c)
            sc_old = jnp.exp(m_old - m_new)
            sc_c = jnp.exp(m_c - m_new)
            m_acc[h] = m_new
            l_acc[h] = l_acc[h] * sc_old + l_c * sc_c
            o_acc[h] = o_acc[h] * sc_old + o_c * sc_c

        @pl.when(c == NSTEPS - 1)
        def _():
            left = lax.rem(my_pos - 1 + N_DEV, N_DEV)
            right = lax.rem(my_pos + 1, N_DEV)

            packed = jnp.concatenate(
                [o_acc[...], m_acc[...], l_acc[...],
                 jnp.zeros((H, B, PACK - D - 2), jnp.float32)],
                axis=-1,
            )
            gather_ref[my_pos] = packed

            barrier_sem = pltpu.get_barrier_semaphore()
            for nbr in (left, right):
                pl.semaphore_signal(
                    barrier_sem, inc=1,
                    device_id=(nbr,), device_id_type=pl.DeviceIdType.MESH,
                )
            pl.semaphore_wait(barrier_sem, 2)

            for h in range(N_DEV - 1):
                slot = lax.rem(my_pos - h + N_DEV, N_DEV)
                rdma = pltpu.make_async_remote_copy(
                    src_ref=gather_ref.at[slot],
                    dst_ref=gather_ref.at[slot],
                    send_sem=send_sems.at[h],
                    recv_sem=recv_sems.at[h],
                    device_id=(right,),
                    device_id_type=pl.DeviceIdType.MESH,
                )
                rdma.start()
                rdma.wait()

            g = gather_ref[...]
            o_all = g[:, :, :, :D]
            m_all = g[:, :, :, D]
            l_all = g[:, :, :, D + 1]
            m_g = jnp.max(m_all, axis=0)
            coef = jnp.exp(m_all - m_g[None])
            l_tot = jnp.sum(coef * l_all, axis=0)
            o_tot = jnp.sum(coef[:, :, :, None] * o_all, axis=0)
            res = o_tot / l_tot[:, :, None]
            out_ref[...] = res.transpose(1, 0, 2)[:, None, :, :]

    return pl.pallas_call(
        body,
        grid=(NSTEPS,),
        out_shape=jax.ShapeDtypeStruct((B, 1, H, D), jnp.float32),
        in_specs=[
            pl.BlockSpec((B, 1, H, D), lambda c: (0, 0, 0, 0)),
            pl.BlockSpec((CHUNK, BS, H, D), lambda c: (c, 0, 0, 0)),
            pl.BlockSpec((CHUNK, BS, H, D), lambda c: (c, 0, 0, 0)),
            pl.BlockSpec((B, NPAGES), lambda c: (0, 0)),
            pl.BlockSpec((B, 1), lambda c: (0, 0)),
        ],
        out_specs=pl.BlockSpec((B, 1, H, D), lambda c: (0, 0, 0, 0)),
        scratch_shapes=[
            pltpu.VMEM((H, B, D), jnp.float32),
            pltpu.VMEM((H, B, 1), jnp.float32),
            pltpu.VMEM((H, B, 1), jnp.float32),
            pltpu.VMEM((N_DEV, H, B, PACK), jnp.float32),
            pltpu.SemaphoreType.DMA((N_DEV - 1,)),
            pltpu.SemaphoreType.DMA((N_DEV - 1,)),
        ],
        compiler_params=pltpu.CompilerParams(
            collective_id=0, vmem_limit_bytes=56 * 1024 * 1024
        ),
    )(Q, K, V, bt, lens2)


# device time: 48865 ns/iter; 4.0088x vs baseline; 1.1073x over previous
import jax
import jax.numpy as jnp
from jax import lax
from jax.experimental import pallas as pl
from jax.experimental.pallas import tpu as pltpu

N_DEV = 4
B = 8
H = 8
D = 128
BS = 16
NPAGES = 512
CHUNK = 128
NSTEPS = NPAGES // CHUNK
CKEYS = CHUNK * BS
PACK = 256


def kernel(Q, K, V, bt, lens):
    lens2 = lens.reshape(B, 1)

    def body(q_ref, k_ref, v_ref, bt_ref, lens_ref, out_ref,
             o_acc, m_acc, l_acc, gather_ref, send_sems, recv_sems):
        c = pl.program_id(0)
        my_pos = lax.axis_index("i")

        @pl.when(c == 0)
        def _():
            o_acc[...] = jnp.zeros((H, B, D), jnp.float32)
            m_acc[...] = jnp.full((H, B, 1), -1e30, jnp.float32)
            l_acc[...] = jnp.zeros((H, B, 1), jnp.float32)

        bt_v = bt_ref[...]
        lens_v = lens_ref[...]
        base = my_pos * NPAGES + c * CHUNK

        j_idx = lax.broadcasted_iota(jnp.int32, (B, NPAGES), 1)
        valid = (j_idx < lens_v).astype(jnp.float32)
        local_id = bt_v - base
        p_iota = lax.broadcasted_iota(jnp.int32, (B, NPAGES, CHUNK), 2)
        onehot = (local_id[:, :, None] == p_iota).astype(jnp.float32)
        counts = jnp.sum(valid[:, :, None] * onehot, axis=1)
        counts_key = jnp.broadcast_to(
            counts[:, :, None], (B, CHUNK, BS)
        ).reshape(B, CKEYS)

        qf = q_ref[...].reshape(B, H * D)
        kf = k_ref[...].reshape(CKEYS, H * D)
        vf = v_ref[...].reshape(CKEYS, H * D)
        scale = D ** -0.5

        for h in range(H):
            q_h = qf[:, h * D:(h + 1) * D].astype(jnp.bfloat16)
            k_h = kf[:, h * D:(h + 1) * D].astype(jnp.bfloat16)
            v_h = vf[:, h * D:(h + 1) * D].astype(jnp.bfloat16)

            s_h = lax.dot_general(
                q_h, k_h, (((1,), (1,)), ((), ())),
                preferred_element_type=jnp.float32,
            ) * scale
            s_m = jnp.where(counts_key > 0, s_h, -1e30)
            m_c = jnp.max(s_m, axis=-1, keepdims=True)
            w = counts_key * jnp.exp(s_m - m_c)
            l_c = jnp.sum(w, axis=-1, keepdims=True)
            o_c = lax.dot_general(
                w.astype(jnp.bfloat16), v_h, (((1,), (0,)), ((), ())),
                preferred_element_type=jnp.float32,
            )

            m_old = m_acc[h]
            m_new = jnp.maximum(m_old, m_c)
            sc_old = jnp.exp(m_old - m_new)
            sc_c = jnp.exp(m_c - m_new)
            m_acc[h] = m_new
            l_acc[h] = l_acc[h] * sc_old + l_c * sc_c
            o_acc[h] = o_acc[h] * sc_old + o_c * sc_c

        @pl.when(c == NSTEPS - 1)
        def _():
            left = lax.rem(my_pos - 1 + N_DEV, N_DEV)
            right = lax.rem(my_pos + 1, N_DEV)

            packed = jnp.concatenate(
                [o_acc[...], m_acc[...], l_acc[...],
                 jnp.zeros((H, B, PACK - D - 2), jnp.float32)],
                axis=-1,
            )
            gather_ref[my_pos] = packed

            barrier_sem = pltpu.get_barrier_semaphore()
            for nbr in (left, right):
                pl.semaphore_signal(
                    barrier_sem, inc=1,
                    device_id=(nbr,), device_id_type=pl.DeviceIdType.MESH,
                )
            pl.semaphore_wait(barrier_sem, 2)

            for h in range(N_DEV - 1):
                slot = lax.rem(my_pos - h + N_DEV, N_DEV)
                rdma = pltpu.make_async_remote_copy(
                    src_ref=gather_ref.at[slot],
                    dst_ref=gather_ref.at[slot],
                    send_sem=send_sems.at[h],
                    recv_sem=recv_sems.at[h],
                    device_id=(right,),
                    device_id_type=pl.DeviceIdType.MESH,
                )
                rdma.start()
                rdma.wait()

            g = gather_ref[...]
            o_all = g[:, :, :, :D]
            m_all = g[:, :, :, D]
            l_all = g[:, :, :, D + 1]
            m_g = jnp.max(m_all, axis=0)
            coef = jnp.exp(m_all - m_g[None])
            l_tot = jnp.sum(coef * l_all, axis=0)
            o_tot = jnp.sum(coef[:, :, :, None] * o_all, axis=0)
            res = o_tot / l_tot[:, :, None]
            out_ref[...] = res.transpose(1, 0, 2)[:, None, :, :]

    return pl.pallas_call(
        body,
        grid=(NSTEPS,),
        out_shape=jax.ShapeDtypeStruct((B, 1, H, D), jnp.float32),
        in_specs=[
            pl.BlockSpec((B, 1, H, D), lambda c: (0, 0, 0, 0)),
            pl.BlockSpec((CHUNK, BS, H, D), lambda c: (c, 0, 0, 0)),
            pl.BlockSpec((CHUNK, BS, H, D), lambda c: (c, 0, 0, 0)),
            pl.BlockSpec((B, NPAGES), lambda c: (0, 0)),
            pl.BlockSpec((B, 1), lambda c: (0, 0)),
        ],
        out_specs=pl.BlockSpec((B, 1, H, D), lambda c: (0, 0, 0, 0)),
        scratch_shapes=[
            pltpu.VMEM((H, B, D), jnp.float32),
            pltpu.VMEM((H, B, 1), jnp.float32),
            pltpu.VMEM((H, B, 1), jnp.float32),
            pltpu.VMEM((N_DEV, H, B, PACK), jnp.float32),
            pltpu.SemaphoreType.DMA((N_DEV - 1,)),
            pltpu.SemaphoreType.DMA((N_DEV - 1,)),
        ],
        compiler_params=pltpu.CompilerParams(
            collective_id=0, vmem_limit_bytes=56 * 1024 * 1024
        ),
    )(Q, K, V, bt, lens2)
